# baseline (speedup 1.0000x reference)
import functools

import jax
import jax.numpy as jnp
from jax import lax
from jax.experimental import pallas as pl
from jax.experimental.pallas import tpu as pltpu
from jax.experimental.pallas import tpu_sc as plsc

_L = 16


def _make_sc_kernel(B, D, idx_dtype):
    mesh = plsc.VectorSubcoreMesh(core_axis_name="c", subcore_axis_name="s")

    @functools.partial(
        pl.kernel,
        mesh=mesh,
        compiler_params=pltpu.CompilerParams(use_tc_tiling_on_sc=False),
        out_type=(
            jax.ShapeDtypeStruct((D,), jnp.float32),
            jax.ShapeDtypeStruct((D,), jnp.float32),
        ),
        scratch_types=[
            pltpu.VMEM((_L,), jnp.int32),
            pltpu.VMEM((_L,), jnp.int32),
            pltpu.VMEM((_L,), jnp.int32),
            pltpu.VMEM((_L, D), jnp.float32),
            pltpu.VMEM((D, D), jnp.float32),
            pltpu.VMEM((D, D), jnp.float32),
            pltpu.VMEM((D,), jnp.float32),
            pltpu.SemaphoreType.DMA,
        ],
    )
    def sc_fn(left_h, right_h, loi_h, roi_h, table_h, wenc_h, wneg_h,
              outl_h, outr_h,
              oi_v, idx_v, tok_v, row_v, wenc_v, wneg_v, out_v, sem):
        cid = lax.axis_index("c")
        sid = lax.axis_index("s")
        wid = sid * 2 + cid

        def side(tok_h, oi_h, out_h):
            pltpu.sync_copy(wenc_h, wenc_v)
            pltpu.sync_copy(wneg_h, wneg_v)
            pltpu.sync_copy(oi_h, oi_v)
            xv = oi_v[...]
            x = xv[0]
            neg = x < 0
            ax = jnp.where(neg, -x, x)
            idx = ax - 1
            idx = jnp.where(idx < 0, idx + B, idx)
            idx_v[...] = lax.broadcast(idx, (_L,))
            pltpu.async_copy(tok_h.at[idx_v], tok_v, sem).wait()
            pltpu.async_copy(table_h.at[tok_v], row_v, sem).wait()

            nc = D // _L
            ev = [row_v[0, pl.ds(c * _L, _L)] for c in range(nc)]
            acc = [jnp.zeros((_L,), jnp.float32) for _ in range(nc)]
            for k in range(D):
                s = ev[k // _L][k % _L]
                for c in range(nc):
                    acc[c] = acc[c] + s * wenc_v[k, pl.ds(c * _L, _L)]
            th = []
            for c in range(nc):
                t = 1.0 - 2.0 / (jnp.exp(2.0 * acc[c]) + 1.0)
                th.append(t)
            acc2 = [jnp.zeros((_L,), jnp.float32) for _ in range(nc)]
            for k in range(D):
                s2 = th[k // _L][k % _L]
                for c in range(nc):
                    acc2[c] = acc2[c] + s2 * wneg_v[k, pl.ds(c * _L, _L)]
            nf = lax.convert_element_type(neg, jnp.float32)
            pf = 1.0 - nf
            for c in range(nc):
                out_v[pl.ds(c * _L, _L)] = pf * th[c] + nf * acc2[c]
            pltpu.sync_copy(out_v, out_h)

        @pl.when(wid == 0)
        def _():
            side(left_h, loi_h, outl_h)

        @pl.when(wid == 1)
        def _():
            side(right_h, roi_h, outr_h)

    return sc_fn


def kernel(left, right, left_out_ind, right_out_ind, table, W_enc, W_neg):
    B = left.shape[0]
    D = table.shape[1]
    left = left.astype(jnp.int32)
    right = right.astype(jnp.int32)
    loi = jnp.broadcast_to(left_out_ind.astype(jnp.int32), (_L,))
    roi = jnp.broadcast_to(right_out_ind.astype(jnp.int32), (_L,))
    sc_fn = _make_sc_kernel(B, D, jnp.int32)
    out_l, out_r = sc_fn(left, right, loi, roi, table, W_enc, W_neg)
    return out_l, out_r


# speedup vs baseline: 1.7240x; 1.7240x over previous
import functools

import jax
import jax.numpy as jnp
from jax import lax
from jax.experimental import pallas as pl
from jax.experimental.pallas import tpu as pltpu
from jax.experimental.pallas import tpu_sc as plsc

_L = 16


def _make_sc_kernel(B, D):
    mesh = plsc.VectorSubcoreMesh(core_axis_name="c", subcore_axis_name="s")

    @functools.partial(
        pl.kernel,
        mesh=mesh,
        compiler_params=pltpu.CompilerParams(needs_layout_passes=False),
        out_type=(
            jax.ShapeDtypeStruct((D,), jnp.float32),
            jax.ShapeDtypeStruct((D,), jnp.float32),
        ),
        scratch_types=[
            pltpu.VMEM((_L,), jnp.int32),
            pltpu.VMEM((_L,), jnp.int32),
            pltpu.VMEM((8, D), jnp.float32),
            pltpu.VMEM((D, D), jnp.float32),
            pltpu.VMEM((D, D), jnp.float32),
            pltpu.VMEM((D,), jnp.float32),
        ],
    )
    def sc_fn(left_h, right_h, loi_h, roi_h, table_h, wenc_h, wneg_h,
              outl_h, outr_h,
              oi_v, tokbuf_v, rowbuf_v, wenc_v, wneg_v, out_v):
        cid = lax.axis_index("c")
        sid = lax.axis_index("s")
        wid = sid * 2 + cid

        def side(tok_h, oi_h, out_h):
            pltpu.sync_copy(wenc_h, wenc_v)
            pltpu.sync_copy(wneg_h, wneg_v)
            pltpu.sync_copy(oi_h, oi_v)
            x = oi_v[...][0]
            neg = x < 0
            ax = jnp.where(neg, -x, x)
            idx = ax - 1
            idx = jnp.where(idx < 0, idx + B, idx)
            base = pl.multiple_of((idx >> 4) << 4, _L)
            pltpu.sync_copy(tok_h.at[pl.ds(base, _L)], tokbuf_v)
            lane_vec = lax.broadcast(idx - base, (_L,))
            tok_vec = plsc.load_gather(tokbuf_v, [lane_vec])
            token = tok_vec[0]
            big = pl.multiple_of((token >> 3) << 3, 8)
            pltpu.sync_copy(table_h.at[pl.ds(big, 8), :], rowbuf_v)
            rvec = lax.broadcast(token - big, (_L,))
            col = lax.iota(jnp.int32, _L)
            nc = D // _L
            ev = [plsc.load_gather(rowbuf_v, [rvec, col + c * _L])
                  for c in range(nc)]
            acc = [jnp.zeros((_L,), jnp.float32) for _ in range(nc)]
            for k in range(D):
                s = ev[k // _L][k % _L]
                for c in range(nc):
                    acc[c] = acc[c] + s * wenc_v[k, pl.ds(c * _L, _L)]
            th = []
            for c in range(nc):
                th.append(1.0 - 2.0 / (jnp.exp(2.0 * acc[c]) + 1.0))
            acc2 = [jnp.zeros((_L,), jnp.float32) for _ in range(nc)]
            for k in range(D):
                s2 = th[k // _L][k % _L]
                for c in range(nc):
                    acc2[c] = acc2[c] + s2 * wneg_v[k, pl.ds(c * _L, _L)]
            nf = lax.convert_element_type(neg, jnp.float32)
            pf = 1.0 - nf
            for c in range(nc):
                out_v[pl.ds(c * _L, _L)] = pf * th[c] + nf * acc2[c]
            pltpu.sync_copy(out_v, out_h)

        @pl.when(wid == 0)
        def _():
            side(left_h, loi_h, outl_h)

        @pl.when(wid == 1)
        def _():
            side(right_h, roi_h, outr_h)

    return sc_fn


def kernel(left, right, left_out_ind, right_out_ind, table, W_enc, W_neg):
    B = left.shape[0]
    D = table.shape[1]
    left = left.astype(jnp.int32)
    right = right.astype(jnp.int32)
    loi = jnp.broadcast_to(left_out_ind.astype(jnp.int32), (_L,))
    roi = jnp.broadcast_to(right_out_ind.astype(jnp.int32), (_L,))
    sc_fn = _make_sc_kernel(B, D)
    out_l, out_r = sc_fn(left, right, loi, roi, table, W_enc, W_neg)
    return out_l, out_r


# speedup vs baseline: 23.8248x; 13.8198x over previous
import functools

import jax
import jax.numpy as jnp
from jax import lax
from jax.experimental import pallas as pl
from jax.experimental.pallas import tpu as pltpu
from jax.experimental.pallas import tpu_sc as plsc

_L = 16


def _make_sc_kernel(B, D):
    mesh = plsc.VectorSubcoreMesh(core_axis_name="c", subcore_axis_name="s")

    @functools.partial(
        pl.kernel,
        mesh=mesh,
        compiler_params=pltpu.CompilerParams(
            needs_layout_passes=False,
            disable_bounds_checks=True,
        ),
        out_type=(
            jax.ShapeDtypeStruct((D,), jnp.float32),
            jax.ShapeDtypeStruct((D,), jnp.float32),
        ),
        scratch_types=[
            pltpu.VMEM((_L,), jnp.int32),
            pltpu.VMEM((_L,), jnp.int32),
            pltpu.VMEM((D, 128), jnp.float32),
            pltpu.VMEM((D, D), jnp.float32),
            pltpu.VMEM((D, D), jnp.float32),
            pltpu.VMEM((D,), jnp.float32),
        ],
    )
    def sc_fn(left_h, right_h, loi_h, roi_h, tableT_h, wenc_h, wneg_h,
              outl_h, outr_h,
              oi_v, tokbuf_v, colbuf_v, wenc_v, wneg_v, out_v):
        cid = lax.axis_index("c")
        sid = lax.axis_index("s")
        wid = sid * 2 + cid

        def side(tok_h, oi_h, out_h):
            pltpu.sync_copy(wenc_h, wenc_v)
            pltpu.sync_copy(wneg_h, wneg_v)
            pltpu.sync_copy(oi_h, oi_v)
            x = oi_v[...][0]
            neg = x < 0
            ax = jnp.where(neg, -x, x)
            idx = ax - 1
            idx = jnp.where(idx < 0, idx + B, idx)
            base = pl.multiple_of((idx >> 4) << 4, _L)
            pltpu.sync_copy(tok_h.at[pl.ds(base, _L)], tokbuf_v)
            lane_vec = lax.broadcast(idx - base, (_L,))
            tok_vec = plsc.load_gather(tokbuf_v, [lane_vec])
            token = tok_vec[0]
            cbase = pl.multiple_of((token >> 7) << 7, 128)
            pltpu.sync_copy(tableT_h.at[:, pl.ds(cbase, 128)], colbuf_v)
            cvec = lax.broadcast(token - cbase, (_L,))
            row_iota = lax.iota(jnp.int32, _L)
            nc = D // _L
            ev = [plsc.load_gather(colbuf_v, [row_iota + c * _L, cvec])
                  for c in range(nc)]
            acc = [jnp.zeros((_L,), jnp.float32) for _ in range(nc)]
            for k in range(D):
                s = ev[k // _L][k % _L]
                for c in range(nc):
                    acc[c] = acc[c] + s * wenc_v[k, pl.ds(c * _L, _L)]
            th = []
            for c in range(nc):
                th.append(1.0 - 2.0 / (jnp.exp(2.0 * acc[c]) + 1.0))
            acc2 = [jnp.zeros((_L,), jnp.float32) for _ in range(nc)]
            for k in range(D):
                s2 = th[k // _L][k % _L]
                for c in range(nc):
                    acc2[c] = acc2[c] + s2 * wneg_v[k, pl.ds(c * _L, _L)]
            nf = lax.convert_element_type(neg, jnp.float32)
            pf = 1.0 - nf
            for c in range(nc):
                out_v[pl.ds(c * _L, _L)] = pf * th[c] + nf * acc2[c]
            pltpu.sync_copy(out_v, out_h)

        @pl.when(wid == 0)
        def _():
            side(left_h, loi_h, outl_h)

        @pl.when(wid == 1)
        def _():
            side(right_h, roi_h, outr_h)

    return sc_fn


def kernel(left, right, left_out_ind, right_out_ind, table, W_enc, W_neg):
    B = left.shape[0]
    D = table.shape[1]
    left = left.astype(jnp.int32)
    right = right.astype(jnp.int32)
    loi = jnp.broadcast_to(left_out_ind.astype(jnp.int32), (_L,))
    roi = jnp.broadcast_to(right_out_ind.astype(jnp.int32), (_L,))
    sc_fn = _make_sc_kernel(B, D)
    out_l, out_r = sc_fn(left, right, loi, roi, table.T, W_enc, W_neg)
    return out_l, out_r


# speedup vs baseline: 24.8050x; 1.0411x over previous
import functools

import jax
import jax.numpy as jnp
from jax import lax
from jax.experimental import pallas as pl
from jax.experimental.pallas import tpu as pltpu
from jax.experimental.pallas import tpu_sc as plsc

_L = 16


def _make_sc_kernel(B, D):
    mesh = plsc.VectorSubcoreMesh(core_axis_name="c", subcore_axis_name="s")

    @functools.partial(
        pl.kernel,
        mesh=mesh,
        compiler_params=pltpu.CompilerParams(
            needs_layout_passes=False,
            disable_bounds_checks=True,
        ),
        out_type=(
            jax.ShapeDtypeStruct((D,), jnp.float32),
            jax.ShapeDtypeStruct((D,), jnp.float32),
        ),
        scratch_types=[
            pltpu.VMEM((1,), jnp.int32),
            pltpu.VMEM((_L,), jnp.int32),
            pltpu.VMEM((D, 128), jnp.float32),
            pltpu.VMEM((D, D), jnp.float32),
            pltpu.VMEM((D, D), jnp.float32),
            pltpu.VMEM((D,), jnp.float32),
            pltpu.SemaphoreType.DMA,
        ],
    )
    def sc_fn(left_h, right_h, loi_h, roi_h, tableT_h, wenc_h, wneg_h,
              outl_h, outr_h,
              oi_v, tokbuf_v, colbuf_v, wenc_v, wneg_v, out_v, wsem):
        cid = lax.axis_index("c")
        sid = lax.axis_index("s")
        wid = sid * 2 + cid

        def side(tok_h, oi_h, out_h):
            wenc_cp = pltpu.make_async_copy(wenc_h, wenc_v, wsem)
            wenc_cp.start()
            pltpu.sync_copy(oi_h, oi_v)
            zero16 = lax.broadcast(jnp.int32(0), (_L,))
            x = plsc.load_gather(oi_v, [zero16])[0]
            neg = x < 0
            ax = jnp.where(neg, -x, x)
            idx = ax - 1
            idx = jnp.where(idx < 0, idx + B, idx)
            base = pl.multiple_of((idx >> 4) << 4, _L)
            pltpu.sync_copy(tok_h.at[pl.ds(base, _L)], tokbuf_v)
            lane_vec = lax.broadcast(idx - base, (_L,))
            tok_vec = plsc.load_gather(tokbuf_v, [lane_vec])
            token = tok_vec[0]
            cbase = pl.multiple_of((token >> 7) << 7, 128)
            pltpu.sync_copy(tableT_h.at[:, pl.ds(cbase, 128)], colbuf_v)
            cvec = lax.broadcast(token - cbase, (_L,))
            row_iota = lax.iota(jnp.int32, _L)
            nc = D // _L
            ev = [plsc.load_gather(colbuf_v, [row_iota + c * _L, cvec])
                  for c in range(nc)]
            wenc_cp.wait()
            acc = [jnp.zeros((_L,), jnp.float32) for _ in range(nc)]
            for k in range(D):
                s = ev[k // _L][k % _L]
                for c in range(nc):
                    acc[c] = acc[c] + s * wenc_v[k, pl.ds(c * _L, _L)]
            th = []
            for c in range(nc):
                t = 1.0 - 2.0 / (jnp.exp(2.0 * acc[c]) + 1.0)
                th.append(t)
                out_v[pl.ds(c * _L, _L)] = t
            @pl.when(neg)
            def _():
                pltpu.sync_copy(wneg_h, wneg_v)
                acc2 = [jnp.zeros((_L,), jnp.float32) for _ in range(nc)]
                for k in range(D):
                    s2 = th[k // _L][k % _L]
                    for c in range(nc):
                        acc2[c] = acc2[c] + s2 * wneg_v[k, pl.ds(c * _L, _L)]
                for c in range(nc):
                    out_v[pl.ds(c * _L, _L)] = acc2[c]
            pltpu.sync_copy(out_v, out_h)

        @pl.when(wid == 0)
        def _():
            side(left_h, loi_h, outl_h)

        @pl.when(wid == 1)
        def _():
            side(right_h, roi_h, outr_h)

    return sc_fn


def kernel(left, right, left_out_ind, right_out_ind, table, W_enc, W_neg):
    B = left.shape[0]
    D = table.shape[1]
    left = left.astype(jnp.int32)
    right = right.astype(jnp.int32)
    loi = left_out_ind.astype(jnp.int32)
    roi = right_out_ind.astype(jnp.int32)
    sc_fn = _make_sc_kernel(B, D)
    out_l, out_r = sc_fn(left, right, loi, roi, table.T, W_enc, W_neg)
    return out_l, out_r


# speedup vs baseline: 26.3929x; 1.0640x over previous
import functools

import jax
import jax.numpy as jnp
from jax import lax
from jax.experimental import pallas as pl
from jax.experimental.pallas import tpu as pltpu
from jax.experimental.pallas import tpu_sc as plsc

_L = 16


def _make_sc_kernel(B, D):
    mesh = plsc.VectorSubcoreMesh(core_axis_name="c", subcore_axis_name="s",
                                  num_cores=1)

    @functools.partial(
        pl.kernel,
        mesh=mesh,
        compiler_params=pltpu.CompilerParams(
            needs_layout_passes=False,
            disable_bounds_checks=True,
        ),
        out_type=(
            jax.ShapeDtypeStruct((D,), jnp.float32),
            jax.ShapeDtypeStruct((D,), jnp.float32),
        ),
        scratch_types=[
            pltpu.VMEM((1,), jnp.int32),
            pltpu.VMEM((_L,), jnp.int32),
            pltpu.VMEM((D, 128), jnp.float32),
            pltpu.VMEM((D, D), jnp.float32),
            pltpu.VMEM((D, D), jnp.float32),
            pltpu.VMEM((D,), jnp.float32),
            pltpu.SemaphoreType.DMA,
        ],
    )
    def sc_fn(left_h, right_h, loi_h, roi_h, tableT_h, wenc_h, wneg_h,
              outl_h, outr_h,
              oi_v, tokbuf_v, colbuf_v, wenc_v, wneg_v, out_v, wsem):
        wid = lax.axis_index("s")

        def side(tok_h, oi_h, out_h):
            wenc_cp = pltpu.make_async_copy(wenc_h, wenc_v, wsem)
            wenc_cp.start()
            pltpu.sync_copy(oi_h, oi_v)
            zero16 = lax.broadcast(jnp.int32(0), (_L,))
            x = plsc.load_gather(oi_v, [zero16])[0]
            neg = x < 0
            ax = jnp.where(neg, -x, x)
            idx = ax - 1
            idx = jnp.where(idx < 0, idx + B, idx)
            base = pl.multiple_of((idx >> 4) << 4, _L)
            pltpu.sync_copy(tok_h.at[pl.ds(base, _L)], tokbuf_v)
            lane_vec = lax.broadcast(idx - base, (_L,))
            tok_vec = plsc.load_gather(tokbuf_v, [lane_vec])
            token = tok_vec[0]
            cbase = pl.multiple_of((token >> 7) << 7, 128)
            pltpu.sync_copy(tableT_h.at[:, pl.ds(cbase, 128)], colbuf_v)
            cvec = lax.broadcast(token - cbase, (_L,))
            row_iota = lax.iota(jnp.int32, _L)
            nc = D // _L
            ev = [plsc.load_gather(colbuf_v, [row_iota + c * _L, cvec])
                  for c in range(nc)]
            wenc_cp.wait()
            acc = [jnp.zeros((_L,), jnp.float32) for _ in range(nc)]
            for k in range(D):
                s = ev[k // _L][k % _L]
                for c in range(nc):
                    acc[c] = acc[c] + s * wenc_v[k, pl.ds(c * _L, _L)]
            th = []
            for c in range(nc):
                t = 1.0 - 2.0 / (jnp.exp(2.0 * acc[c]) + 1.0)
                th.append(t)
                out_v[pl.ds(c * _L, _L)] = t
            @pl.when(neg)
            def _():
                pltpu.sync_copy(wneg_h, wneg_v)
                acc2 = [jnp.zeros((_L,), jnp.float32) for _ in range(nc)]
                for k in range(D):
                    s2 = th[k // _L][k % _L]
                    for c in range(nc):
                        acc2[c] = acc2[c] + s2 * wneg_v[k, pl.ds(c * _L, _L)]
                for c in range(nc):
                    out_v[pl.ds(c * _L, _L)] = acc2[c]
            pltpu.sync_copy(out_v, out_h)

        @pl.when(wid == 0)
        def _():
            side(left_h, loi_h, outl_h)

        @pl.when(wid == 1)
        def _():
            side(right_h, roi_h, outr_h)

    return sc_fn


def kernel(left, right, left_out_ind, right_out_ind, table, W_enc, W_neg):
    B = left.shape[0]
    D = table.shape[1]
    left = left.astype(jnp.int32)
    right = right.astype(jnp.int32)
    loi = left_out_ind.astype(jnp.int32)
    roi = right_out_ind.astype(jnp.int32)
    sc_fn = _make_sc_kernel(B, D)
    out_l, out_r = sc_fn(left, right, loi, roi, table.T, W_enc, W_neg)
    return out_l, out_r


# speedup vs baseline: 27.7132x; 1.0500x over previous
import functools

import jax
import jax.numpy as jnp
from jax import lax
from jax.experimental import pallas as pl
from jax.experimental.pallas import tpu as pltpu
from jax.experimental.pallas import tpu_sc as plsc

_L = 16


def _make_sc_kernel(B, D):
    mesh = plsc.VectorSubcoreMesh(core_axis_name="c", subcore_axis_name="s",
                                  num_cores=1)

    @functools.partial(
        pl.kernel,
        mesh=mesh,
        compiler_params=pltpu.CompilerParams(
            needs_layout_passes=False,
            disable_bounds_checks=True,
        ),
        out_type=(
            jax.ShapeDtypeStruct((D,), jnp.float32),
            jax.ShapeDtypeStruct((D,), jnp.float32),
        ),
        scratch_types=[
            pltpu.VMEM((1,), jnp.int32),
            pltpu.VMEM((_L,), jnp.int32),
            pltpu.VMEM((D, 128), jnp.float32),
            pltpu.VMEM((D, D), jnp.float32),
            pltpu.VMEM((D, D), jnp.float32),
            pltpu.VMEM((D,), jnp.float32),
            pltpu.VMEM((D,), jnp.float32),
            pltpu.SemaphoreType.DMA,
        ],
    )
    def sc_fn(left_h, right_h, loi_h, roi_h, tableT_h, wenc_h, wneg_h,
              outl_h, outr_h,
              oi_v, tokbuf_v, colbuf_v, wenc_v, wneg_v, emb_v, out_v, wsem):
        wid = lax.axis_index("s")

        def side(tok_h, oi_h, out_h):
            wenc_cp = pltpu.make_async_copy(wenc_h, wenc_v, wsem)
            wenc_cp.start()
            pltpu.sync_copy(oi_h, oi_v)
            zero16 = lax.broadcast(jnp.int32(0), (_L,))
            x = plsc.load_gather(oi_v, [zero16])[0]
            neg = x < 0
            ax = jnp.where(neg, -x, x)
            idx = ax - 1
            idx = jnp.where(idx < 0, idx + B, idx)
            base = pl.multiple_of((idx >> 4) << 4, _L)
            pltpu.sync_copy(tok_h.at[pl.ds(base, _L)], tokbuf_v)
            lane_vec = lax.broadcast(idx - base, (_L,))
            tok_vec = plsc.load_gather(tokbuf_v, [lane_vec])
            token = tok_vec[0]
            cbase = pl.multiple_of((token >> 7) << 7, 128)
            pltpu.sync_copy(tableT_h.at[:, pl.ds(cbase, 128)], colbuf_v)
            cvec = lax.broadcast(token - cbase, (_L,))
            row_iota = lax.iota(jnp.int32, _L)
            nc = D // _L
            for c in range(nc):
                emb_v[pl.ds(c * _L, _L)] = plsc.load_gather(
                    colbuf_v, [row_iota + c * _L, cvec])

            def gemv(src_v, w_v):
                def body(g, accs):
                    vec = src_v[pl.ds(g * _L, _L)]
                    for l in range(_L):
                        s = vec[l]
                        row = g * _L + l
                        accs = tuple(
                            accs[c] + s * w_v[row, pl.ds(c * _L, _L)]
                            for c in range(nc))
                    return accs
                zero = jnp.zeros((_L,), jnp.float32)
                return lax.fori_loop(0, D // _L, body, (zero,) * nc,
                                     unroll=False)

            wenc_cp.wait()
            acc = gemv(emb_v, wenc_v)
            for c in range(nc):
                out_v[pl.ds(c * _L, _L)] = (
                    1.0 - 2.0 / (jnp.exp(2.0 * acc[c]) + 1.0))
            @pl.when(neg)
            def _():
                pltpu.sync_copy(wneg_h, wneg_v)
                acc2 = gemv(out_v, wneg_v)
                for c in range(nc):
                    out_v[pl.ds(c * _L, _L)] = acc2[c]
            pltpu.sync_copy(out_v, out_h)

        @pl.when(wid == 0)
        def _():
            side(left_h, loi_h, outl_h)

        @pl.when(wid == 1)
        def _():
            side(right_h, roi_h, outr_h)

    return sc_fn


def kernel(left, right, left_out_ind, right_out_ind, table, W_enc, W_neg):
    B = left.shape[0]
    D = table.shape[1]
    left = left.astype(jnp.int32)
    right = right.astype(jnp.int32)
    loi = left_out_ind.astype(jnp.int32)
    roi = right_out_ind.astype(jnp.int32)
    sc_fn = _make_sc_kernel(B, D)
    out_l, out_r = sc_fn(left, right, loi, roi, table.T, W_enc, W_neg)
    return out_l, out_r


# speedup vs baseline: 27.7752x; 1.0022x over previous
import functools

import jax
import jax.numpy as jnp
from jax import lax
from jax.experimental import pallas as pl
from jax.experimental.pallas import tpu as pltpu
from jax.experimental.pallas import tpu_sc as plsc

_L = 16


def _make_sc_kernel(B, D):
    mesh = plsc.VectorSubcoreMesh(core_axis_name="c", subcore_axis_name="s",
                                  num_cores=1, num_subcores=2)

    @functools.partial(
        pl.kernel,
        mesh=mesh,
        compiler_params=pltpu.CompilerParams(
            needs_layout_passes=False,
            disable_bounds_checks=True,
        ),
        out_type=(
            jax.ShapeDtypeStruct((D,), jnp.float32),
            jax.ShapeDtypeStruct((D,), jnp.float32),
        ),
        scratch_types=[
            pltpu.VMEM((1,), jnp.int32),
            pltpu.VMEM((_L,), jnp.int32),
            pltpu.VMEM((D, 128), jnp.float32),
            pltpu.VMEM((D, D), jnp.float32),
            pltpu.VMEM((D, D), jnp.float32),
            pltpu.VMEM((D,), jnp.float32),
            pltpu.VMEM((D,), jnp.float32),
            pltpu.SemaphoreType.DMA,
        ],
    )
    def sc_fn(left_h, right_h, loi_h, roi_h, tableT_h, wenc_h, wneg_h,
              outl_h, outr_h,
              oi_v, tokbuf_v, colbuf_v, wenc_v, wneg_v, emb_v, out_v, wsem):
        wid = lax.axis_index("s")

        def side(tok_h, oi_h, out_h):
            wenc_cp = pltpu.make_async_copy(wenc_h, wenc_v, wsem)
            wenc_cp.start()
            pltpu.sync_copy(oi_h, oi_v)
            zero16 = lax.broadcast(jnp.int32(0), (_L,))
            x = plsc.load_gather(oi_v, [zero16])[0]
            neg = x < 0
            ax = jnp.where(neg, -x, x)
            idx = ax - 1
            idx = jnp.where(idx < 0, idx + B, idx)
            base = pl.multiple_of((idx >> 4) << 4, _L)
            pltpu.sync_copy(tok_h.at[pl.ds(base, _L)], tokbuf_v)
            lane_vec = lax.broadcast(idx - base, (_L,))
            tok_vec = plsc.load_gather(tokbuf_v, [lane_vec])
            token = tok_vec[0]
            cbase = pl.multiple_of((token >> 7) << 7, 128)
            pltpu.sync_copy(tableT_h.at[:, pl.ds(cbase, 128)], colbuf_v)
            cvec = lax.broadcast(token - cbase, (_L,))
            row_iota = lax.iota(jnp.int32, _L)
            nc = D // _L
            for c in range(nc):
                emb_v[pl.ds(c * _L, _L)] = plsc.load_gather(
                    colbuf_v, [row_iota + c * _L, cvec])

            def gemv(src_v, w_v):
                def body(g, accs):
                    vec = src_v[pl.ds(g * _L, _L)]
                    for l in range(_L):
                        s = vec[l]
                        row = g * _L + l
                        accs = tuple(
                            accs[c] + s * w_v[row, pl.ds(c * _L, _L)]
                            for c in range(nc))
                    return accs
                zero = jnp.zeros((_L,), jnp.float32)
                return lax.fori_loop(0, D // _L, body, (zero,) * nc,
                                     unroll=False)

            wenc_cp.wait()
            acc = gemv(emb_v, wenc_v)
            for c in range(nc):
                out_v[pl.ds(c * _L, _L)] = (
                    1.0 - 2.0 / (jnp.exp(2.0 * acc[c]) + 1.0))
            @pl.when(neg)
            def _():
                pltpu.sync_copy(wneg_h, wneg_v)
                acc2 = gemv(out_v, wneg_v)
                for c in range(nc):
                    out_v[pl.ds(c * _L, _L)] = acc2[c]
            pltpu.sync_copy(out_v, out_h)

        @pl.when(wid == 0)
        def _():
            side(left_h, loi_h, outl_h)

        @pl.when(wid == 1)
        def _():
            side(right_h, roi_h, outr_h)

    return sc_fn


def kernel(left, right, left_out_ind, right_out_ind, table, W_enc, W_neg):
    B = left.shape[0]
    D = table.shape[1]
    left = left.astype(jnp.int32)
    right = right.astype(jnp.int32)
    loi = left_out_ind.astype(jnp.int32)
    roi = right_out_ind.astype(jnp.int32)
    sc_fn = _make_sc_kernel(B, D)
    out_l, out_r = sc_fn(left, right, loi, roi, table.T, W_enc, W_neg)
    return out_l, out_r
